# baseline (speedup 1.0000x reference)
import functools

import jax
import jax.numpy as jnp
from jax import lax
from jax.experimental import pallas as pl
from jax.experimental.pallas import tpu as pltpu
from jax.experimental.pallas import tpu_sc as plsc

C = 128
K = 32
NC = 2
NS = 16
NW = NC * NS
L = 16
G = 8
NBUF = 2


def _pool_body(table_hbm, idx_hbm, out_hbm, idx_v, rows_v, outv_v, gsems,
               osems):
    pts_per_w = out_hbm.shape[0] // NW
    n_groups = pts_per_w // G
    wid = lax.axis_index("s") * NC + lax.axis_index("c")
    base_pt = wid * pts_per_w

    pltpu.sync_copy(idx_hbm.at[pl.ds(base_pt * K, pts_per_w * K)], idx_v)

    def gather(g, b):
        return pltpu.make_async_copy(
            table_hbm.at[idx_v.at[pl.ds(g * (G * K), G * K)]],
            rows_v.at[b], gsems.at[b])

    def put(g, b):
        return pltpu.make_async_copy(
            outv_v.at[b], out_hbm.at[pl.ds(base_pt + g * G, G)], osems.at[b])

    for b in range(NBUF):
        gather(b, b).start()

    def pair(i, _):
        g0 = i * NBUF
        for b in range(NBUF):
            g = g0 + b
            gather(g, b).wait()

            @pl.when(i > 0)
            def _():
                put(g - NBUF, b).wait()

            for p in range(G):
                def row_step(r, accs, p=p, b=b):
                    return tuple(
                        jnp.maximum(accs[f],
                                    rows_v[b, p * K + r, pl.ds(f * L, L)])
                        for f in range(C // L))

                accs0 = tuple(rows_v[b, p * K, pl.ds(f * L, L)]
                              for f in range(C // L))
                accs = lax.fori_loop(0, K, row_step, accs0, unroll=4)
                for f in range(C // L):
                    outv_v[b, p, pl.ds(f * L, L)] = accs[f]

            put(g, b).start()

            @pl.when(g + NBUF < n_groups)
            def _():
                gather(g + NBUF, b).start()

        return ()

    lax.fori_loop(0, n_groups // NBUF, pair, (), unroll=False)
    for b in range(NBUF):
        put(n_groups - NBUF + b, b).wait()


def kernel(inputs, nn_count, nn_index):
    mp = nn_count.shape[0]
    pts_per_w = ((mp + NW * G * NBUF - 1) // (NW * G * NBUF)) * (G * NBUF)
    mp_pad = NW * pts_per_w

    idx = nn_index.astype(jnp.int32)
    count = nn_count.astype(jnp.int32)
    mask = jnp.arange(K, dtype=jnp.int32)[None, :] < count[:, None]
    idx_dup = jnp.where(mask, idx, idx[:, :1])
    idx_flat = jnp.zeros((mp_pad, K), jnp.int32).at[:mp].set(idx_dup)
    idx_flat = idx_flat.reshape(mp_pad * K)

    grid_kernel = pl.kernel(
        _pool_body,
        out_type=jax.ShapeDtypeStruct((mp_pad, C), jnp.float32),
        mesh=plsc.VectorSubcoreMesh(core_axis_name="c", subcore_axis_name="s"),
        scratch_types=[
            pltpu.VMEM((pts_per_w * K,), jnp.int32),
            pltpu.VMEM((NBUF, G * K, C), jnp.float32),
            pltpu.VMEM((NBUF, G, C), jnp.float32),
            pltpu.SemaphoreType.DMA((NBUF,)),
            pltpu.SemaphoreType.DMA((NBUF,)),
        ],
    )
    out = grid_kernel(inputs, idx_flat)
    return out[:mp]


# speedup vs baseline: 1.0069x; 1.0069x over previous
import functools

import jax
import jax.numpy as jnp
from jax import lax
from jax.experimental import pallas as pl
from jax.experimental.pallas import tpu as pltpu
from jax.experimental.pallas import tpu_sc as plsc

C = 128
K = 32
NC = 2
NS = 16
NW = NC * NS
L = 16
G = 4
NBUF = 4


def _pool_body(table_hbm, idx_hbm, out_hbm, idx_v, rows_v, outv_v, gsems,
               osems):
    pts_per_w = out_hbm.shape[0] // NW
    n_groups = pts_per_w // G
    wid = lax.axis_index("s") * NC + lax.axis_index("c")
    base_pt = wid * pts_per_w

    pltpu.sync_copy(idx_hbm.at[pl.ds(base_pt * K, pts_per_w * K)], idx_v)

    def gather(g, b):
        return pltpu.make_async_copy(
            table_hbm.at[idx_v.at[pl.ds(g * (G * K), G * K)]],
            rows_v.at[b], gsems.at[b])

    def put(g, b):
        return pltpu.make_async_copy(
            outv_v.at[b], out_hbm.at[pl.ds(base_pt + g * G, G)], osems.at[b])

    for b in range(NBUF):
        gather(b, b).start()

    def ring(i, _):
        g0 = i * NBUF
        for b in range(NBUF):
            g = g0 + b
            gather(g, b).wait()

            @pl.when(i > 0)
            def _():
                put(g - NBUF, b).wait()

            for p in range(G):
                def row_step(r, accs, p=p, b=b):
                    return tuple(
                        jnp.maximum(accs[f],
                                    rows_v[b, p * K + r, pl.ds(f * L, L)])
                        for f in range(C // L))

                accs0 = tuple(rows_v[b, p * K, pl.ds(f * L, L)]
                              for f in range(C // L))
                accs = lax.fori_loop(0, K, row_step, accs0, unroll=4)
                for f in range(C // L):
                    outv_v[b, p, pl.ds(f * L, L)] = accs[f]

            put(g, b).start()

            @pl.when(g + NBUF < n_groups)
            def _():
                gather(g + NBUF, b).start()

        return ()

    lax.fori_loop(0, n_groups // NBUF, ring, (), unroll=False)
    for b in range(NBUF):
        put(n_groups - NBUF + b, b).wait()


def kernel(inputs, nn_count, nn_index):
    mp = nn_count.shape[0]
    pts_per_w = ((mp + NW * G * NBUF - 1) // (NW * G * NBUF)) * (G * NBUF)
    mp_pad = NW * pts_per_w

    idx = nn_index.astype(jnp.int32)
    count = nn_count.astype(jnp.int32)
    mask = jnp.arange(K, dtype=jnp.int32)[None, :] < count[:, None]
    idx_dup = jnp.where(mask, idx, idx[:, :1])
    idx_flat = jnp.zeros((mp_pad, K), jnp.int32).at[:mp].set(idx_dup)
    idx_flat = idx_flat.reshape(mp_pad * K)

    grid_kernel = pl.kernel(
        _pool_body,
        out_type=jax.ShapeDtypeStruct((mp_pad, C), jnp.float32),
        mesh=plsc.VectorSubcoreMesh(core_axis_name="c", subcore_axis_name="s"),
        scratch_types=[
            pltpu.VMEM((pts_per_w * K,), jnp.int32),
            pltpu.VMEM((NBUF, G * K, C), jnp.float32),
            pltpu.VMEM((NBUF, G, C), jnp.float32),
            pltpu.SemaphoreType.DMA((NBUF,)),
            pltpu.SemaphoreType.DMA((NBUF,)),
        ],
    )
    out = grid_kernel(inputs, idx_flat)
    return out[:mp]
